# baseline (speedup 1.0000x reference)
import functools

import jax
import jax.numpy as jnp
from jax import lax
from jax.experimental import pallas as pl
from jax.experimental.pallas import tpu as pltpu
from jax.experimental.pallas import tpu_sc as plsc

WINDOW = 8
NC = 2
NS = 16
LANES = 16


def _splat(ref, i):
    return plsc.load_gather(ref, [jnp.full((LANES,), i, jnp.int32)])


def _make_sc_call(B, L, C, H, W):
    HW = H * W
    P = min(512, HW)
    G = min(128, P)
    CAP = P + LANES
    mesh = plsc.VectorSubcoreMesh(core_axis_name="c", subcore_axis_name="s",
                                  num_cores=NC, num_subcores=NS)

    @functools.partial(
        pl.kernel,
        out_type=jax.ShapeDtypeStruct((B * L, HW, C), jnp.float32),
        mesh=mesh,
        compiler_params=pltpu.CompilerParams(
            needs_layout_passes=False, use_tc_tiling_on_sc=False),
        scratch_types=[
            pltpu.VMEM((P,), jnp.float32),
            pltpu.VMEM((P,), jnp.float32),
            pltpu.VMEM((P,), jnp.float32),
            pltpu.VMEM((P,), jnp.float32),
            [pltpu.VMEM((CAP,), jnp.int32) for _ in range(4)],
            [pltpu.VMEM((CAP,), jnp.float32) for _ in range(4)],
            [pltpu.VMEM((CAP,), jnp.int32) for _ in range(4)],
            pltpu.VMEM((4 * P, C), jnp.float32),
            pltpu.VMEM((P, C), jnp.float32),
            pltpu.VMEM((LANES,), jnp.float32),
            pltpu.SemaphoreType.DMA,
        ],
    )
    def sc_call(cpb_hbm, cum_hbm, img_hbm, wk_hbm, out_hbm,
                glx, gly, ckx, cky, idxbs, wcbs, pcbs, rowsb, acc, wkv, sem):
        cid = lax.axis_index("c")
        sid = lax.axis_index("s")
        f = sid * NC + cid
        b = f // L
        l = f % L
        k0 = jnp.maximum(l - (WINDOW - 1), 0)
        pltpu.sync_copy(wk_hbm, wkv)
        iota16 = lax.iota(jnp.int32, LANES)
        zero16i = jnp.zeros((LANES,), jnp.int32)
        zero16f = jnp.zeros((LANES,), jnp.float32)

        def zidx_body(z, _):
            z0 = pl.multiple_of(z * LANES, LANES)
            for t in range(4):
                idxbs[t][pl.ds(z0, LANES)] = zero16i
            return 0
        lax.fori_loop(0, CAP // LANES, zidx_body, 0)

        def chunk_body(ci, _):
            c0 = pl.multiple_of(ci * P, P)
            pltpu.sync_copy(cpb_hbm.at[f, 0, pl.ds(c0, P)], glx)
            pltpu.sync_copy(cpb_hbm.at[f, 1, pl.ds(c0, P)], gly)
            pltpu.sync_copy(img_hbm.at[pl.ds(f * HW + c0, P), :], acc)

            def k_body(k, _):
                bk = b * L + k
                pltpu.sync_copy(cum_hbm.at[bk, 0, pl.ds(c0, P)], ckx)
                pltpu.sync_copy(cum_hbm.at[bk, 1, pl.ds(c0, P)], cky)
                wkd = _splat(wkv, l - k)
                rowbase = bk * HW
                ns = [jnp.int32(0)] * 4

                for g in range(P // LANES):
                    s = pl.ds(g * LANES, LANES)
                    gx = glx[s] - ckx[s]
                    gy = gly[s] - cky[s]
                    ix = (gx + 1.0) * (W * 0.5) - 0.5
                    iy = (gy + 1.0) * (H * 0.5) - 0.5
                    xt = ix.astype(jnp.int32)
                    x0 = xt - (xt.astype(jnp.float32) > ix).astype(jnp.int32)
                    fx = ix - x0.astype(jnp.float32)
                    yt = iy.astype(jnp.int32)
                    y0 = yt - (yt.astype(jnp.float32) > iy).astype(jnp.int32)
                    fy = iy - y0.astype(jnp.float32)
                    x1 = x0 + 1
                    y1 = y0 + 1
                    vx0 = (x0 >= 0) & (x0 < W)
                    vx1 = (x1 >= 0) & (x1 < W)
                    vy0 = (y0 >= 0) & (y0 < H)
                    vy1 = (y1 >= 0) & (y1 < H)
                    ofx = 1.0 - fx
                    ofy = 1.0 - fy
                    piota = iota16 + g * LANES
                    rx0 = rowbase + y0 * W + x0
                    taps = (
                        (vx0 & vy0, wkd * (ofx * ofy), rx0),
                        (vx0 & vy1, wkd * (ofx * fy), rx0 + W),
                        (vx1 & vy0, wkd * (fx * ofy), rx0 + 1),
                        (vx1 & vy1, wkd * (fx * fy), rx0 + W + 1),
                    )
                    for t, (mask, wt, it) in enumerate(taps):
                        n = ns[t]
                        plsc.store_compressed(idxbs[t].at[pl.ds(n, LANES)],
                                              it, mask=mask)
                        plsc.store_compressed(wcbs[t].at[pl.ds(n, LANES)],
                                              wt, mask=mask)
                        plsc.store_compressed(pcbs[t].at[pl.ds(n, LANES)],
                                              piota, mask=mask)
                        ns[t] = n + jnp.sum(mask.astype(jnp.int32), axis=0)

                for t in range(4):
                    wcbs[t][pl.ds(ns[t], LANES)] = zero16f
                    pcbs[t][pl.ds(ns[t], LANES)] = zero16i

                cps = []
                for t in range(4):
                    for j in range(P // G):
                        @pl.when(j * G < ns[t])
                        def _(t=t, j=j):
                            cps.append(pltpu.async_copy(
                                img_hbm.at[idxbs[t].at[pl.ds(j * G, G)]],
                                rowsb.at[pl.ds(t * P + j * G, G), :], sem))
                for t in range(4):
                    for j in range(P // G):
                        @pl.when(j * G < ns[t])
                        def _(t=t, j=j):
                            cps[0].wait()
                            del cps[0]

                for t in range(4):
                    def blk_body(blk, _, t=t):
                        i0 = blk * LANES
                        wv = wcbs[t][pl.ds(i0, LANES)]
                        pv = pcbs[t][pl.ds(i0, LANES)]
                        src = iota16 + (t * P + i0)
                        for c in range(C):
                            fc = jnp.full((LANES,), c, jnp.int32)
                            r = plsc.load_gather(rowsb, [src, fc])
                            plsc.addupdate_scatter(acc, [pv, fc], wv * r)
                        return 0
                    nblk = (ns[t] + LANES - 1) // LANES
                    lax.fori_loop(0, nblk, blk_body, 0)
                return 0

            lax.fori_loop(k0, l, k_body, 0)
            pltpu.sync_copy(acc, out_hbm.at[f, pl.ds(c0, P), :])
            return 0

        lax.fori_loop(0, HW // P, chunk_body, 0)

    return sc_call


def kernel(flows, images, decay_log):
    B, L, C, H, W = images.shape
    HW = H * W
    cum = jnp.cumsum(flows.astype(jnp.float32), axis=1)
    gx = jnp.linspace(-1.0 + 1.0 / W, 1.0 - 1.0 / W, W)
    gy = jnp.linspace(-1.0 + 1.0 / H, 1.0 - 1.0 / H, H)
    mx, my = jnp.meshgrid(gx, gy, indexing="xy")
    base = jnp.stack([mx, my], axis=0).astype(jnp.float32)
    cpb = cum + base[None, None]
    cum2 = cum.reshape(B * L, 2, HW)
    cpb2 = cpb.reshape(B * L, 2, HW)
    imgflat = (images.astype(jnp.float32)
               .transpose(0, 1, 3, 4, 2)
               .reshape(B * L * HW, C))
    decay = jnp.exp(decay_log)
    dist = jnp.arange(LANES, dtype=jnp.float32)
    wks = jnp.exp(-decay * dist)
    out = _make_sc_call(B, L, C, H, W)(cpb2, cum2, imgflat, wks)
    out = out.reshape(B, L, H, W, C).transpose(0, 1, 4, 2, 3)
    return out.astype(images.dtype)


# speedup vs baseline: 2.8152x; 2.8152x over previous
import functools

import jax
import jax.numpy as jnp
from jax import lax
from jax.experimental import pallas as pl
from jax.experimental.pallas import tpu as pltpu
from jax.experimental.pallas import tpu_sc as plsc

WINDOW = 8
NC = 2
NS = 16
LANES = 16


def _splat(ref, i):
    return plsc.load_gather(ref, [jnp.full((LANES,), i, jnp.int32)])


def _make_sc_call(B, L, C, H, W):
    HW = H * W
    P = min(512, HW)
    G = min(128, P)
    CAP = 4 * P + 2 * LANES
    NSTREAM = (4 * P) // G
    mesh = plsc.VectorSubcoreMesh(core_axis_name="c", subcore_axis_name="s",
                                  num_cores=NC, num_subcores=NS)

    @functools.partial(
        pl.kernel,
        out_type=jax.ShapeDtypeStruct((B * L, HW, C), jnp.float32),
        mesh=mesh,
        compiler_params=pltpu.CompilerParams(
            needs_layout_passes=False, use_tc_tiling_on_sc=False),
        scratch_types=[
            pltpu.VMEM((P,), jnp.float32),
            pltpu.VMEM((P,), jnp.float32),
            pltpu.VMEM((WINDOW, 2, P), jnp.float32),
            pltpu.VMEM((CAP,), jnp.int32),
            pltpu.VMEM((CAP,), jnp.float32),
            pltpu.VMEM((CAP,), jnp.int32),
            pltpu.VMEM((4 * P, C), jnp.float32),
            pltpu.VMEM((P, C), jnp.float32),
            pltpu.VMEM((LANES,), jnp.float32),
            pltpu.SemaphoreType.DMA,
        ],
    )
    def sc_call(cpb_hbm, cum_hbm, img_hbm, wk_hbm, out_hbm,
                glx, gly, ckb, idxb, wcb, pcb, rowsb, acc, wkv, sem):
        cid = lax.axis_index("c")
        sid = lax.axis_index("s")
        f = sid * NC + cid
        b = f // L
        l = f % L
        k0 = jnp.maximum(l - (WINDOW - 1), 0)
        pltpu.sync_copy(wk_hbm, wkv)
        iota16 = lax.iota(jnp.int32, LANES)
        zero16i = jnp.zeros((LANES,), jnp.int32)
        zero16f = jnp.zeros((LANES,), jnp.float32)

        def zidx_body(z, _):
            z0 = pl.multiple_of(z * LANES, LANES)
            idxb[pl.ds(z0, LANES)] = zero16i
            return 0
        lax.fori_loop(0, CAP // LANES, zidx_body, 0)

        def chunk_body(ci, _):
            c0 = pl.multiple_of(ci * P, P)
            pltpu.sync_copy(cpb_hbm.at[f, 0, pl.ds(c0, P)], glx)
            pltpu.sync_copy(cpb_hbm.at[f, 1, pl.ds(c0, P)], gly)
            pltpu.sync_copy(
                cum_hbm.at[pl.ds(b * L + k0, WINDOW), :, pl.ds(c0, P)], ckb)
            pltpu.sync_copy(img_hbm.at[pl.ds(f * HW + c0, P), :], acc)

            def k_body(k, _):
                bk = b * L + k
                kk = k - k0
                wkd = _splat(wkv, l - k)
                rowbase = bk * HW
                n = jnp.int32(0)

                for g in range(P // LANES):
                    s = pl.ds(g * LANES, LANES)
                    ix = glx[s] - ckb[kk, 0, s]
                    iy = gly[s] - ckb[kk, 1, s]
                    xt = ix.astype(jnp.int32)
                    x0 = xt - (xt.astype(jnp.float32) > ix).astype(jnp.int32)
                    fx = ix - x0.astype(jnp.float32)
                    yt = iy.astype(jnp.int32)
                    y0 = yt - (yt.astype(jnp.float32) > iy).astype(jnp.int32)
                    fy = iy - y0.astype(jnp.float32)
                    x1 = x0 + 1
                    y1 = y0 + 1
                    vx0 = x0.astype(jnp.uint32) < W
                    vx1 = x1.astype(jnp.uint32) < W
                    vy0 = y0.astype(jnp.uint32) < H
                    vy1 = y1.astype(jnp.uint32) < H
                    ofx = 1.0 - fx
                    ofy = 1.0 - fy
                    piota = iota16 + g * LANES
                    rx0 = rowbase + y0 * W + x0
                    taps = (
                        (vx0 & vy0, wkd * (ofx * ofy), rx0),
                        (vx0 & vy1, wkd * (ofx * fy), rx0 + W),
                        (vx1 & vy0, wkd * (fx * ofy), rx0 + 1),
                        (vx1 & vy1, wkd * (fx * fy), rx0 + W + 1),
                    )
                    for mask, wt, it in taps:
                        plsc.store_compressed(idxb.at[pl.ds(n, LANES)],
                                              it, mask=mask)
                        plsc.store_compressed(wcb.at[pl.ds(n, LANES)],
                                              wt, mask=mask)
                        plsc.store_compressed(pcb.at[pl.ds(n, LANES)],
                                              piota, mask=mask)
                        n = n + jnp.sum(mask.astype(jnp.int32), axis=0)

                wcb[pl.ds(n, LANES)] = zero16f
                pcb[pl.ds(n, LANES)] = zero16i

                cps = []
                for j in range(NSTREAM):
                    @pl.when(j * G < n)
                    def _(j=j):
                        cps.append(pltpu.async_copy(
                            img_hbm.at[idxb.at[pl.ds(j * G, G)]],
                            rowsb.at[pl.ds(j * G, G), :], sem))
                for j in range(NSTREAM):
                    @pl.when(j * G < n)
                    def _(j=j):
                        cps[0].wait()
                        del cps[0]

                def blk_body(blk, _):
                    i0 = blk * LANES
                    wv16 = wcb[pl.ds(i0, LANES)]
                    pv16 = pcb[pl.ds(i0, LANES)]
                    for r in range(LANES):
                        i = i0 + r
                        wv = jnp.full((LANES,), wv16[r])
                        p = pv16[r]
                        plsc.addupdate(acc.at[p, pl.ds(0, LANES)],
                                       wv * rowsb[i, pl.ds(0, LANES)])
                        plsc.addupdate(acc.at[p, pl.ds(LANES, LANES)],
                                       wv * rowsb[i, pl.ds(LANES, LANES)])
                    return 0
                nblk = (n + LANES - 1) // LANES
                lax.fori_loop(0, nblk, blk_body, 0)
                return 0

            lax.fori_loop(k0, l, k_body, 0)
            pltpu.sync_copy(acc, out_hbm.at[f, pl.ds(c0, P), :])
            return 0

        lax.fori_loop(0, HW // P, chunk_body, 0)

    return sc_call


def kernel(flows, images, decay_log):
    B, L, C, H, W = images.shape
    HW = H * W
    cum = jnp.cumsum(flows.astype(jnp.float32), axis=1)
    gx = jnp.linspace(-1.0 + 1.0 / W, 1.0 - 1.0 / W, W)
    gy = jnp.linspace(-1.0 + 1.0 / H, 1.0 - 1.0 / H, H)
    mx, my = jnp.meshgrid(gx, gy, indexing="xy")
    base = jnp.stack([mx, my], axis=0).astype(jnp.float32)
    scale = jnp.array([W * 0.5, H * 0.5], jnp.float32).reshape(1, 1, 2, 1, 1)
    av = (cum + base[None, None] + 1.0) * scale - 0.5
    bv = cum * scale
    cpb2 = av.reshape(B * L, 2, HW)
    cum2 = bv.reshape(B * L, 2, HW)
    imgflat = (images.astype(jnp.float32)
               .transpose(0, 1, 3, 4, 2)
               .reshape(B * L * HW, C))
    decay = jnp.exp(decay_log)
    dist = jnp.arange(LANES, dtype=jnp.float32)
    wks = jnp.exp(-decay * dist)
    out = _make_sc_call(B, L, C, H, W)(cpb2, cum2, imgflat, wks)
    out = out.reshape(B, L, H, W, C).transpose(0, 1, 4, 2, 3)
    return out.astype(images.dtype)


# speedup vs baseline: 2.8644x; 1.0175x over previous
import functools

import jax
import jax.numpy as jnp
from jax import lax
from jax.experimental import pallas as pl
from jax.experimental.pallas import tpu as pltpu
from jax.experimental.pallas import tpu_sc as plsc

WINDOW = 8
NC = 2
NS = 16
LANES = 16


def _splat(ref, i):
    return plsc.load_gather(ref, [jnp.full((LANES,), i, jnp.int32)])


def _make_sc_call(B, L, C, H, W):
    HW = H * W
    P = min(512, HW)
    G = min(128, P)
    CAP = 4 * P + 2 * LANES
    NSTREAM = (4 * P) // G
    mesh = plsc.VectorSubcoreMesh(core_axis_name="c", subcore_axis_name="s",
                                  num_cores=NC, num_subcores=NS)

    @functools.partial(
        pl.kernel,
        out_type=jax.ShapeDtypeStruct((B * L, HW, C), jnp.float32),
        mesh=mesh,
        compiler_params=pltpu.CompilerParams(
            needs_layout_passes=False, use_tc_tiling_on_sc=False),
        scratch_types=[
            pltpu.VMEM((P,), jnp.float32),
            pltpu.VMEM((P,), jnp.float32),
            pltpu.VMEM((WINDOW, 2, P), jnp.float32),
            pltpu.VMEM((CAP,), jnp.int32),
            pltpu.VMEM((CAP,), jnp.float32),
            pltpu.VMEM((CAP,), jnp.int32),
            pltpu.VMEM((4 * P, C), jnp.float32),
            pltpu.VMEM((P, C), jnp.float32),
            pltpu.VMEM((LANES,), jnp.float32),
            pltpu.SemaphoreType.DMA,
        ],
    )
    def sc_call(cpb_hbm, cum_hbm, img_hbm, wk_hbm, out_hbm,
                glx, gly, ckb, idxb, wcb, pcb, rowsb, acc, wkv, sem):
        cid = lax.axis_index("c")
        sid = lax.axis_index("s")
        f = sid * NC + cid
        b = f // L
        l = f % L
        k0 = jnp.maximum(l - (WINDOW - 1), 0)
        pltpu.sync_copy(wk_hbm, wkv)
        iota16 = lax.iota(jnp.int32, LANES)
        zero16i = jnp.zeros((LANES,), jnp.int32)
        zero16f = jnp.zeros((LANES,), jnp.float32)

        def zidx_body(z, _):
            z0 = pl.multiple_of(z * LANES, LANES)
            idxb[pl.ds(z0, LANES)] = zero16i
            return 0
        lax.fori_loop(0, CAP // LANES, zidx_body, 0)

        def chunk_body(ci, _):
            c0 = pl.multiple_of(ci * P, P)
            pltpu.sync_copy(cpb_hbm.at[f, 0, pl.ds(c0, P)], glx)
            pltpu.sync_copy(cpb_hbm.at[f, 1, pl.ds(c0, P)], gly)
            pltpu.sync_copy(
                cum_hbm.at[pl.ds(b * L + k0, WINDOW), :, pl.ds(c0, P)], ckb)
            pltpu.sync_copy(img_hbm.at[pl.ds(f * HW + c0, P), :], acc)

            def k_body(k, _):
                bk = b * L + k
                kk = k - k0
                wkd = _splat(wkv, l - k)
                rowbase = bk * HW
                n = jnp.int32(0)

                for g in range(P // LANES):
                    s = pl.ds(g * LANES, LANES)
                    ix = glx[s] - ckb[kk, 0, s]
                    iy = gly[s] - ckb[kk, 1, s]
                    xt = ix.astype(jnp.int32)
                    x0 = xt - (xt.astype(jnp.float32) > ix).astype(jnp.int32)
                    fx = ix - x0.astype(jnp.float32)
                    yt = iy.astype(jnp.int32)
                    y0 = yt - (yt.astype(jnp.float32) > iy).astype(jnp.int32)
                    fy = iy - y0.astype(jnp.float32)
                    x1 = x0 + 1
                    y1 = y0 + 1
                    vx0 = x0.astype(jnp.uint32) < W
                    vx1 = x1.astype(jnp.uint32) < W
                    vy0 = y0.astype(jnp.uint32) < H
                    vy1 = y1.astype(jnp.uint32) < H
                    ofx = 1.0 - fx
                    ofy = 1.0 - fy
                    piota = iota16 + g * LANES
                    rx0 = rowbase + y0 * W + x0
                    taps = (
                        (vx0 & vy0, wkd * (ofx * ofy), rx0),
                        (vx0 & vy1, wkd * (ofx * fy), rx0 + W),
                        (vx1 & vy0, wkd * (fx * ofy), rx0 + 1),
                        (vx1 & vy1, wkd * (fx * fy), rx0 + W + 1),
                    )
                    for mask, wt, it in taps:
                        plsc.store_compressed(idxb.at[pl.ds(n, LANES)],
                                              it, mask=mask)
                        plsc.store_compressed(wcb.at[pl.ds(n, LANES)],
                                              wt, mask=mask)
                        plsc.store_compressed(pcb.at[pl.ds(n, LANES)],
                                              piota, mask=mask)
                        n = n + plsc.all_reduce_population_count(mask)[0]

                wcb[pl.ds(n, LANES)] = zero16f
                pcb[pl.ds(n, LANES)] = zero16i

                cps = []
                for j in range(NSTREAM):
                    @pl.when(j * G < n)
                    def _(j=j):
                        cps.append(pltpu.async_copy(
                            img_hbm.at[idxb.at[pl.ds(j * G, G)]],
                            rowsb.at[pl.ds(j * G, G), :], sem))
                for j in range(NSTREAM):
                    @pl.when(j * G < n)
                    def _(j=j):
                        cps[0].wait()
                        del cps[0]

                def blk_body(blk, _):
                    i0 = blk * LANES
                    wv16 = wcb[pl.ds(i0, LANES)]
                    pv16 = pcb[pl.ds(i0, LANES)]
                    for r in range(LANES):
                        i = i0 + r
                        wv = jnp.full((LANES,), wv16[r])
                        p = pv16[r]
                        plsc.addupdate(acc.at[p, pl.ds(0, LANES)],
                                       wv * rowsb[i, pl.ds(0, LANES)])
                        plsc.addupdate(acc.at[p, pl.ds(LANES, LANES)],
                                       wv * rowsb[i, pl.ds(LANES, LANES)])
                    return 0
                nblk = (n + LANES - 1) // LANES
                lax.fori_loop(0, nblk, blk_body, 0)
                return 0

            lax.fori_loop(k0, l, k_body, 0)
            pltpu.sync_copy(acc, out_hbm.at[f, pl.ds(c0, P), :])
            return 0

        lax.fori_loop(0, HW // P, chunk_body, 0)

    return sc_call


def kernel(flows, images, decay_log):
    B, L, C, H, W = images.shape
    HW = H * W
    cum = jnp.cumsum(flows.astype(jnp.float32), axis=1)
    gx = jnp.linspace(-1.0 + 1.0 / W, 1.0 - 1.0 / W, W)
    gy = jnp.linspace(-1.0 + 1.0 / H, 1.0 - 1.0 / H, H)
    mx, my = jnp.meshgrid(gx, gy, indexing="xy")
    base = jnp.stack([mx, my], axis=0).astype(jnp.float32)
    scale = jnp.array([W * 0.5, H * 0.5], jnp.float32).reshape(1, 1, 2, 1, 1)
    av = (cum + base[None, None] + 1.0) * scale - 0.5
    bv = cum * scale
    cpb2 = av.reshape(B * L, 2, HW)
    cum2 = bv.reshape(B * L, 2, HW)
    imgflat = (images.astype(jnp.float32)
               .transpose(0, 1, 3, 4, 2)
               .reshape(B * L * HW, C))
    decay = jnp.exp(decay_log)
    dist = jnp.arange(LANES, dtype=jnp.float32)
    wks = jnp.exp(-decay * dist)
    out = _make_sc_call(B, L, C, H, W)(cpb2, cum2, imgflat, wks)
    out = out.reshape(B, L, H, W, C).transpose(0, 1, 4, 2, 3)
    return out.astype(images.dtype)
